# baseline (speedup 1.0000x reference)
import functools

import jax
import jax.numpy as jnp
from jax import lax
from jax.experimental import pallas as pl
from jax.experimental.pallas import tpu as pltpu
from jax.experimental.pallas import tpu_sc as plsc

S = 3
L = 50
B = 16384
D = 64
R = S * B
NC, NS = 2, 16
NW = NC * NS
TILE_ROWS = R // NW
ROWS_PER_STEP = 2
G = ROWS_PER_STEP * L
STEPS = TILE_ROWS // ROWS_PER_STEP
OUT_BLK_STEPS = 128
OUT_BLK = OUT_BLK_STEPS * ROWS_PER_STEP
NBLK = STEPS // OUT_BLK_STEPS
NBUF = 4

_sc_mesh = plsc.VectorSubcoreMesh(
    core_axis_name="c", subcore_axis_name="s", num_cores=NC, num_subcores=NS
)


@functools.partial(
    pl.kernel,
    out_type=jax.ShapeDtypeStruct((R, D), jnp.float32),
    mesh=_sc_mesh,
    scratch_types=[
        pltpu.VMEM((STEPS, G), jnp.int32),
        [pltpu.VMEM((G, D), jnp.float32) for _ in range(NBUF)],
        pltpu.VMEM((OUT_BLK, D), jnp.float32),
        [pltpu.SemaphoreType.DMA for _ in range(NBUF)],
    ],
    compiler_params=pltpu.CompilerParams(use_tc_tiling_on_sc=False),
)
def _sc_pool(idx_hbm, emb_hbm, out_hbm, idxbuf, rowbufs, outbuf, sems):
    wid = lax.axis_index("s") * NC + lax.axis_index("c")
    pltpu.sync_copy(idx_hbm.at[wid], idxbuf)
    for b in range(NBUF):
        pltpu.async_copy(emb_hbm.at[idxbuf.at[b]], rowbufs[b], sems[b])

    def accum(rows, j):
        for c in range(ROWS_PER_STEP):
            accs = [rows[L * c, pl.ds(g * 16, 16)] for g in range(D // 16)]
            for l in range(1, L):
                for g in range(D // 16):
                    accs[g] = accs[g] + rows[L * c + l, pl.ds(g * 16, 16)]
            row = (j % OUT_BLK_STEPS) * ROWS_PER_STEP + c
            for g in range(D // 16):
                outbuf[row, pl.ds(g * 16, 16)] = accs[g]

    def pair(jp, carry):
        j0 = jp * NBUF
        for par in range(NBUF):
            rows = rowbufs[par]
            sem = sems[par]
            j = j0 + par
            pltpu.make_async_copy(emb_hbm.at[pl.ds(0, G)], rows, sem).wait()
            accum(rows, j)

            @pl.when(j % OUT_BLK_STEPS == OUT_BLK_STEPS - 1)
            def _flush():
                pltpu.sync_copy(
                    outbuf,
                    out_hbm.at[
                        pl.ds(
                            wid * TILE_ROWS + (j // OUT_BLK_STEPS) * OUT_BLK, OUT_BLK
                        )
                    ],
                )

            @pl.when(j + NBUF < STEPS)
            def _next():
                pltpu.async_copy(emb_hbm.at[idxbuf.at[j + NBUF]], rows, sem)

        return carry

    lax.fori_loop(0, STEPS // NBUF, pair, 0)


def _mlp_body(pool_ref, w1_ref, b1_ref, w2_ref, b2_ref, out_ref):
    f32 = jnp.float32
    h = (
        jnp.dot(pool_ref[0], w1_ref[0:D, :], preferred_element_type=f32)
        + jnp.dot(pool_ref[1], w1_ref[D : 2 * D, :], preferred_element_type=f32)
        + jnp.dot(pool_ref[2], w1_ref[2 * D : 3 * D, :], preferred_element_type=f32)
    )
    h = h * (1.0 / B) + b1_ref[0]
    h = jnp.maximum(h, 0.0)
    o = jnp.dot(h, w2_ref[...], preferred_element_type=f32) + b2_ref[0]
    m = jnp.max(o, axis=1, keepdims=True)
    e = o - m
    out_ref[...] = e - jnp.log(jnp.sum(jnp.exp(e), axis=1, keepdims=True))


def _tc_mlp(pooled3, W1, b1, W2, b2):
    BLK = 512
    grid = (B // BLK,)
    return pl.pallas_call(
        _mlp_body,
        grid=grid,
        in_specs=[
            pl.BlockSpec((S, BLK, D), lambda i: (0, i, 0)),
            pl.BlockSpec((S * D, 128), lambda i: (0, 0)),
            pl.BlockSpec((1, 128), lambda i: (0, 0)),
            pl.BlockSpec((128, 10), lambda i: (0, 0)),
            pl.BlockSpec((1, 10), lambda i: (0, 0)),
        ],
        out_specs=pl.BlockSpec((BLK, 10), lambda i: (i, 0)),
        out_shape=jax.ShapeDtypeStruct((B, 10), jnp.float32),
    )(pooled3, W1, b1, W2, b2)


def kernel(batch_inputs, emb, W1, b1, W2, b2):
    idx = jnp.transpose(batch_inputs, (0, 2, 1)).reshape(NW, STEPS, G)
    pooled = _sc_pool(idx, emb)
    pooled3 = pooled.reshape(S, B, D)
    return _tc_mlp(pooled3, W1, b1.reshape(1, -1), W2, b2.reshape(1, -1))


# speedup vs baseline: 1.1156x; 1.1156x over previous
import functools

import jax
import jax.numpy as jnp
from jax import lax
from jax.experimental import pallas as pl
from jax.experimental.pallas import tpu as pltpu
from jax.experimental.pallas import tpu_sc as plsc

S = 3
L = 50
B = 16384
D = 64
R = S * B
NC, NS = 2, 16
NW = NC * NS
TILE_ROWS = R // NW
ROWS_PER_STEP = 2
G = ROWS_PER_STEP * L
STEPS = TILE_ROWS // ROWS_PER_STEP
OUT_BLK_STEPS = 128
OUT_BLK = OUT_BLK_STEPS * ROWS_PER_STEP
NBLK = STEPS // OUT_BLK_STEPS
NBUF = 2

_sc_mesh = plsc.VectorSubcoreMesh(
    core_axis_name="c", subcore_axis_name="s", num_cores=NC, num_subcores=NS
)


@functools.partial(
    pl.kernel,
    out_type=jax.ShapeDtypeStruct((R, D), jnp.float32),
    mesh=_sc_mesh,
    scratch_types=[
        pltpu.VMEM((STEPS, G), jnp.int32),
        [pltpu.VMEM((G, D), jnp.float32) for _ in range(NBUF)],
        pltpu.VMEM((OUT_BLK, D), jnp.float32),
        [pltpu.SemaphoreType.DMA for _ in range(NBUF)],
    ],
    compiler_params=pltpu.CompilerParams(use_tc_tiling_on_sc=False),
)
def _sc_pool(idx_hbm, emb_hbm, out_hbm, idxbuf, rowbufs, outbuf, sems):
    wid = lax.axis_index("s") * NC + lax.axis_index("c")
    pltpu.sync_copy(idx_hbm.at[wid], idxbuf)
    for b in range(NBUF):
        pltpu.async_copy(emb_hbm.at[idxbuf.at[b]], rowbufs[b], sems[b])

    def accum(rows, j):
        for c in range(ROWS_PER_STEP):
            accs = [rows[L * c, pl.ds(g * 16, 16)] for g in range(D // 16)]
            for l in range(1, L):
                for g in range(D // 16):
                    accs[g] = accs[g] + rows[L * c + l, pl.ds(g * 16, 16)]
            row = (j % OUT_BLK_STEPS) * ROWS_PER_STEP + c
            for g in range(D // 16):
                outbuf[row, pl.ds(g * 16, 16)] = accs[g]

    def pair(jp, carry):
        j0 = jp * NBUF
        for par in range(NBUF):
            rows = rowbufs[par]
            sem = sems[par]
            j = j0 + par
            pltpu.make_async_copy(emb_hbm.at[pl.ds(0, G)], rows, sem).wait()
            accum(rows, j)

            @pl.when(j % OUT_BLK_STEPS == OUT_BLK_STEPS - 1)
            def _flush():
                pltpu.sync_copy(
                    outbuf,
                    out_hbm.at[
                        pl.ds(
                            wid * TILE_ROWS + (j // OUT_BLK_STEPS) * OUT_BLK, OUT_BLK
                        )
                    ],
                )

            @pl.when(j + NBUF < STEPS)
            def _next():
                pltpu.async_copy(emb_hbm.at[idxbuf.at[j + NBUF]], rows, sem)

        return carry

    lax.fori_loop(0, STEPS // NBUF, pair, 0)


def _mlp_body(pool_ref, w1_ref, b1_ref, w2_ref, b2_ref, out_ref):
    f32 = jnp.float32
    h = (
        jnp.dot(pool_ref[0], w1_ref[0:D, :], preferred_element_type=f32)
        + jnp.dot(pool_ref[1], w1_ref[D : 2 * D, :], preferred_element_type=f32)
        + jnp.dot(pool_ref[2], w1_ref[2 * D : 3 * D, :], preferred_element_type=f32)
    )
    h = h * (1.0 / B) + b1_ref[0]
    h = jnp.maximum(h, 0.0)
    o = jnp.dot(h, w2_ref[...], preferred_element_type=f32) + b2_ref[0]
    m = jnp.max(o, axis=1, keepdims=True)
    e = o - m
    out_ref[...] = e - jnp.log(jnp.sum(jnp.exp(e), axis=1, keepdims=True))


def _tc_mlp(pooled3, W1, b1, W2, b2):
    BLK = 512
    grid = (B // BLK,)
    return pl.pallas_call(
        _mlp_body,
        grid=grid,
        in_specs=[
            pl.BlockSpec((S, BLK, D), lambda i: (0, i, 0)),
            pl.BlockSpec((S * D, 128), lambda i: (0, 0)),
            pl.BlockSpec((1, 128), lambda i: (0, 0)),
            pl.BlockSpec((128, 10), lambda i: (0, 0)),
            pl.BlockSpec((1, 10), lambda i: (0, 0)),
        ],
        out_specs=pl.BlockSpec((BLK, 10), lambda i: (i, 0)),
        out_shape=jax.ShapeDtypeStruct((B, 10), jnp.float32),
    )(pooled3, W1, b1, W2, b2)


def kernel(batch_inputs, emb, W1, b1, W2, b2):
    idx = jnp.transpose(batch_inputs, (0, 2, 1)).reshape(NW, STEPS, G)
    pooled = _sc_pool(idx, emb)
    pooled3 = pooled.reshape(S, B, D)
    return _tc_mlp(pooled3, W1, b1.reshape(1, -1), W2, b2.reshape(1, -1))


# speedup vs baseline: 1.4647x; 1.3129x over previous
import functools

import jax
import jax.numpy as jnp
from jax import lax
from jax.experimental import pallas as pl
from jax.experimental.pallas import tpu as pltpu
from jax.experimental.pallas import tpu_sc as plsc

S = 3
L = 50
B = 16384
D = 64
NC, NS = 2, 16
NW = NC * NS
COLS = B // NW
NCH = 4
CH = COLS // NCH
VG = D // 16


@functools.partial(
    pl.kernel,
    out_type=jax.ShapeDtypeStruct((S, B, D), jnp.float32),
    mesh=plsc.VectorSubcoreMesh(
        core_axis_name="c", subcore_axis_name="s", num_cores=NC, num_subcores=NS
    ),
    scratch_types=[
        [pltpu.VMEM((COLS,), jnp.int32) for _ in range(2)],
        [[pltpu.VMEM((CH, D), jnp.float32) for _ in range(NCH)] for _ in range(2)],
        pltpu.VMEM((COLS, D), jnp.float32),
        [pltpu.SemaphoreType.DMA for _ in range(2)],
        [[pltpu.SemaphoreType.DMA for _ in range(NCH)] for _ in range(2)],
    ],
    compiler_params=pltpu.CompilerParams(use_tc_tiling_on_sc=False),
)
def _sc_pool(idx_hbm, emb_hbm, out_hbm, idxbufs, rowbufs, outbuf, isems, gsems):
    wid = lax.axis_index("s") * NC + lax.axis_index("c")
    col0 = wid * COLS

    def issue_gathers(p):
        for k in range(NCH):
            pltpu.async_copy(
                emb_hbm.at[idxbufs[p].at[pl.ds(k * CH, CH)]],
                rowbufs[p][k],
                gsems[p][k],
            )

    def wait_gathers(p):
        for k in range(NCH):
            pltpu.make_async_copy(
                emb_hbm.at[pl.ds(0, CH)], rowbufs[p][k], gsems[p][k]
            ).wait()

    def accum(p, first):
        for k in range(NCH):
            rb = rowbufs[p][k]

            def body(i4, _):
                for u in range(4):
                    i = i4 * 4 + u
                    for g in range(VG):
                        v = rb[i, pl.ds(g * 16, 16)]
                        if first:
                            outbuf[k * CH + i, pl.ds(g * 16, 16)] = v
                        else:
                            plsc.addupdate(
                                outbuf.at[k * CH + i, pl.ds(g * 16, 16)], v
                            )
                return 0

            lax.fori_loop(0, CH // 4, body, 0)

    for s in range(S):
        pltpu.sync_copy(idx_hbm.at[s, 0, pl.ds(col0, COLS)], idxbufs[0])
        pltpu.async_copy(idx_hbm.at[s, 1, pl.ds(col0, COLS)], idxbufs[1], isems[1])
        issue_gathers(0)

        def pair(lp, _):
            for p in range(2):
                l = lp * 2 + p
                wait_gathers(p)

                @pl.when(l + 2 < L)
                def _next_idx():
                    pltpu.async_copy(
                        idx_hbm.at[s, l + 2, pl.ds(col0, COLS)], idxbufs[p], isems[p]
                    )

                @pl.when(l + 1 < L)
                def _next_gather():
                    q = 1 - p
                    pltpu.make_async_copy(
                        idx_hbm.at[s, 0, pl.ds(col0, COLS)], idxbufs[q], isems[q]
                    ).wait()
                    issue_gathers(q)

                if lp == 0 and p == 0:
                    accum(p, True)
                else:
                    accum(p, False)
            return 0

        pair(0, 0)

        def pair_add(lp, _):
            for p in range(2):
                l = lp * 2 + p
                wait_gathers(p)

                @pl.when(l + 2 < L)
                def _next_idx():
                    pltpu.async_copy(
                        idx_hbm.at[s, l + 2, pl.ds(col0, COLS)], idxbufs[p], isems[p]
                    )

                @pl.when(l + 1 < L)
                def _next_gather():
                    q = 1 - p
                    pltpu.make_async_copy(
                        idx_hbm.at[s, 0, pl.ds(col0, COLS)], idxbufs[q], isems[q]
                    ).wait()
                    issue_gathers(q)

                accum(p, False)
            return 0

        lax.fori_loop(1, L // 2, pair_add, 0)

        pltpu.sync_copy(outbuf, out_hbm.at[s, pl.ds(col0, COLS)])


def _mlp_body(pool_ref, w1_ref, b1_ref, w2_ref, b2_ref, out_ref):
    f32 = jnp.float32
    h = (
        jnp.dot(pool_ref[0], w1_ref[0:D, :], preferred_element_type=f32)
        + jnp.dot(pool_ref[1], w1_ref[D : 2 * D, :], preferred_element_type=f32)
        + jnp.dot(pool_ref[2], w1_ref[2 * D : 3 * D, :], preferred_element_type=f32)
    )
    h = h * (1.0 / B) + b1_ref[0]
    h = jnp.maximum(h, 0.0)
    o = jnp.dot(h, w2_ref[...], preferred_element_type=f32) + b2_ref[0]
    m = jnp.max(o, axis=1, keepdims=True)
    e = o - m
    out_ref[...] = e - jnp.log(jnp.sum(jnp.exp(e), axis=1, keepdims=True))


def _tc_mlp(pooled3, W1, b1, W2, b2):
    BLK = 512
    grid = (B // BLK,)
    return pl.pallas_call(
        _mlp_body,
        grid=grid,
        in_specs=[
            pl.BlockSpec((S, BLK, D), lambda i: (0, i, 0)),
            pl.BlockSpec((S * D, 128), lambda i: (0, 0)),
            pl.BlockSpec((1, 128), lambda i: (0, 0)),
            pl.BlockSpec((128, 10), lambda i: (0, 0)),
            pl.BlockSpec((1, 10), lambda i: (0, 0)),
        ],
        out_specs=pl.BlockSpec((BLK, 10), lambda i: (i, 0)),
        out_shape=jax.ShapeDtypeStruct((B, 10), jnp.float32),
    )(pooled3, W1, b1, W2, b2)


def kernel(batch_inputs, emb, W1, b1, W2, b2):
    pooled3 = _sc_pool(batch_inputs, emb)
    return _tc_mlp(pooled3, W1, b1.reshape(1, -1), W2, b2.reshape(1, -1))


# speedup vs baseline: 1.5068x; 1.0288x over previous
import functools

import jax
import jax.numpy as jnp
from jax import lax
from jax.experimental import pallas as pl
from jax.experimental.pallas import tpu as pltpu
from jax.experimental.pallas import tpu_sc as plsc

S = 3
L = 50
B = 16384
D = 64
NC, NS = 2, 16
NW = NC * NS
COLS = B // NW
NCH = 4
CH = COLS // NCH
VG = D // 16


@functools.partial(
    pl.kernel,
    out_type=jax.ShapeDtypeStruct((S, B, D), jnp.float32),
    mesh=plsc.VectorSubcoreMesh(
        core_axis_name="c", subcore_axis_name="s", num_cores=NC, num_subcores=NS
    ),
    scratch_types=[
        [pltpu.VMEM((COLS,), jnp.int32) for _ in range(2)],
        [[pltpu.VMEM((CH, D), jnp.float32) for _ in range(NCH)] for _ in range(2)],
        pltpu.VMEM((COLS, D), jnp.float32),
        [pltpu.SemaphoreType.DMA for _ in range(2)],
        [[pltpu.SemaphoreType.DMA for _ in range(NCH)] for _ in range(2)],
    ],
    compiler_params=pltpu.CompilerParams(use_tc_tiling_on_sc=False),
)
def _sc_pool(idx_hbm, emb_hbm, out_hbm, idxbufs, rowbufs, outbuf, isems, gsems):
    wid = lax.axis_index("s") * NC + lax.axis_index("c")
    col0 = wid * COLS

    def issue_gathers(p):
        for k in range(NCH):
            pltpu.async_copy(
                emb_hbm.at[idxbufs[p].at[pl.ds(k * CH, CH)]],
                rowbufs[p][k],
                gsems[p][k],
            )

    def wait_gathers(p):
        for k in range(NCH):
            pltpu.make_async_copy(
                emb_hbm.at[pl.ds(0, CH)], rowbufs[p][k], gsems[p][k]
            ).wait()

    def accum(p, first):
        for k in range(NCH):
            rb = rowbufs[p][k]

            def body(i4, _):
                for u in range(4):
                    i = i4 * 4 + u
                    for g in range(VG):
                        v = rb[i, pl.ds(g * 16, 16)]
                        if first:
                            outbuf[k * CH + i, pl.ds(g * 16, 16)] = v
                        else:
                            plsc.addupdate(
                                outbuf.at[k * CH + i, pl.ds(g * 16, 16)], v
                            )
                return 0

            lax.fori_loop(0, CH // 4, body, 0)

    for s in range(S):
        pltpu.sync_copy(idx_hbm.at[s, 0, pl.ds(col0, COLS)], idxbufs[0])
        pltpu.async_copy(idx_hbm.at[s, 1, pl.ds(col0, COLS)], idxbufs[1], isems[1])
        issue_gathers(0)

        def pair(lp, _):
            for p in range(2):
                l = lp * 2 + p
                wait_gathers(p)

                @pl.when(l + 2 < L)
                def _next_idx():
                    pltpu.async_copy(
                        idx_hbm.at[s, l + 2, pl.ds(col0, COLS)], idxbufs[p], isems[p]
                    )

                @pl.when(l + 1 < L)
                def _next_gather():
                    q = 1 - p
                    pltpu.make_async_copy(
                        idx_hbm.at[s, 0, pl.ds(col0, COLS)], idxbufs[q], isems[q]
                    ).wait()
                    issue_gathers(q)

                if lp == 0 and p == 0:
                    accum(p, True)
                else:
                    accum(p, False)
            return 0

        pair(0, 0)

        def pair_add(lp, _):
            for p in range(2):
                l = lp * 2 + p
                wait_gathers(p)

                @pl.when(l + 2 < L)
                def _next_idx():
                    pltpu.async_copy(
                        idx_hbm.at[s, l + 2, pl.ds(col0, COLS)], idxbufs[p], isems[p]
                    )

                @pl.when(l + 1 < L)
                def _next_gather():
                    q = 1 - p
                    pltpu.make_async_copy(
                        idx_hbm.at[s, 0, pl.ds(col0, COLS)], idxbufs[q], isems[q]
                    ).wait()
                    issue_gathers(q)

                accum(p, False)
            return 0

        lax.fori_loop(1, L // 2, pair_add, 0)

        pltpu.sync_copy(outbuf, out_hbm.at[s, pl.ds(col0, COLS)])


def _mlp_body(pool_ref, w1_ref, b1_ref, w2_ref, b2_ref, out_ref):
    f32 = jnp.float32
    h = (
        jnp.dot(pool_ref[0], w1_ref[0:D, :], preferred_element_type=f32)
        + jnp.dot(pool_ref[1], w1_ref[D : 2 * D, :], preferred_element_type=f32)
        + jnp.dot(pool_ref[2], w1_ref[2 * D : 3 * D, :], preferred_element_type=f32)
    )
    h = h * (1.0 / B) + b1_ref[0]
    h = jnp.maximum(h, 0.0)
    o = jnp.dot(h, w2_ref[...], preferred_element_type=f32) + b2_ref[0]
    m = jnp.max(o, axis=1, keepdims=True)
    e = o - m
    out_ref[...] = e - jnp.log(jnp.sum(jnp.exp(e), axis=1, keepdims=True))


def _tc_mlp(pooled3, W1, b1, W2, b2):
    BLK = 2048
    grid = (B // BLK,)
    return pl.pallas_call(
        _mlp_body,
        grid=grid,
        in_specs=[
            pl.BlockSpec((S, BLK, D), lambda i: (0, i, 0)),
            pl.BlockSpec((S * D, 128), lambda i: (0, 0)),
            pl.BlockSpec((1, 128), lambda i: (0, 0)),
            pl.BlockSpec((128, 10), lambda i: (0, 0)),
            pl.BlockSpec((1, 10), lambda i: (0, 0)),
        ],
        out_specs=pl.BlockSpec((BLK, 10), lambda i: (i, 0)),
        out_shape=jax.ShapeDtypeStruct((B, 10), jnp.float32),
    )(pooled3, W1, b1, W2, b2)


def kernel(batch_inputs, emb, W1, b1, W2, b2):
    pooled3 = _sc_pool(batch_inputs, emb)
    return _tc_mlp(pooled3, W1, b1.reshape(1, -1), W2, b2.reshape(1, -1))
